# baseline (speedup 1.0000x reference)
import functools

import jax
import jax.numpy as jnp
from jax import lax
from jax.experimental import pallas as pl
from jax.experimental.pallas import tpu as pltpu
from jax.experimental.pallas import tpu_sc as plsc

N = 10000
E = 320000
D = 128
H = 64
NC = 2
NS = 16
CHUNK = 128
EPT = E // NS
NBUF = 4
CPT = 160
IDXC = CPT + NBUF
EPT_PAD = IDXC * CHUNK
NPAD = 10016
RPT = NPAD // NS
OPT = 624
ZCOPIES = (RPT + CHUNK - 1) // CHUNK

_BLK = 2000


def _agg_call(h2, src_p, dst_p):
    mesh = plsc.VectorSubcoreMesh(core_axis_name="c", subcore_axis_name="s")

    @functools.partial(
        pl.kernel,
        mesh=mesh,
        compiler_params=pltpu.CompilerParams(use_tc_tiling_on_sc=False),
        out_type=jax.ShapeDtypeStruct((NC, N, H), jnp.float32),
        scratch_types=[
            pltpu.VMEM((IDXC, CHUNK), jnp.int32),
            pltpu.VMEM((IDXC, CHUNK), jnp.int32),
            [pltpu.VMEM((CHUNK, H), jnp.float32)] * NBUF,
            pltpu.VMEM((CHUNK, H), jnp.float32),
            pltpu.VMEM_SHARED((NPAD, H), jnp.float32),
            [pltpu.SemaphoreType.DMA] * NBUF,
        ],
    )
    def agg_kernel(h_hbm, src_hbm, dst_hbm, out_hbm,
                   src_v, dst_v, rows_bufs, zbuf, agg_sh, sems):
        c = lax.axis_index("c")
        s = lax.axis_index("s")

        pltpu.sync_copy(src_hbm.at[s], src_v)
        pltpu.sync_copy(dst_hbm.at[s], dst_v)

        zero = jnp.zeros((16,), jnp.float32)

        def zrow(i, _):
            for l in range(H // 16):
                zbuf[i, pl.ds(l * 16, 16)] = zero
            return ()

        lax.fori_loop(0, CHUNK, zrow, ())

        zbase = s * RPT
        for k in range(ZCOPIES):
            nrows = min(CHUNK, RPT - k * CHUNK)
            pltpu.sync_copy(zbuf.at[pl.ds(0, nrows)],
                            agg_sh.at[pl.ds(zbase + k * CHUNK, nrows)])
        plsc.subcore_barrier()

        for b in range(NBUF):
            pltpu.async_copy(h_hbm.at[c].at[src_v.at[b]], rows_bufs[b],
                             sems[b])

        def body(g, _):
            base = g * NBUF
            for b in range(NBUF):
                j = base + b
                pltpu.make_async_copy(h_hbm.at[c].at[src_v.at[j]],
                                      rows_bufs[b], sems[b]).wait()
                pltpu.sync_copy(rows_bufs[b], agg_sh.at[dst_v.at[j]],
                                add=True)
                pltpu.async_copy(h_hbm.at[c].at[src_v.at[j + NBUF]],
                                 rows_bufs[b], sems[b])
            return ()

        lax.fori_loop(0, CPT // NBUF, body, ())
        for b in range(NBUF):
            pltpu.make_async_copy(h_hbm.at[c].at[src_v.at[CPT + b]],
                                  rows_bufs[b], sems[b]).wait()
        plsc.subcore_barrier()

        obase = s * OPT
        pltpu.sync_copy(agg_sh.at[pl.ds(obase, OPT)],
                        out_hbm.at[c, pl.ds(obase, OPT)])

        @pl.when(s == NS - 1)
        def _tail():
            pltpu.sync_copy(agg_sh.at[pl.ds(NS * OPT, N - NS * OPT)],
                            out_hbm.at[c, pl.ds(NS * OPT, N - NS * OPT)])

    return agg_kernel(h2, src_p, dst_p)


def _split(o_ref, res):
    o_ref[0] = res[:, :H]
    o_ref[1] = res[:, H:]


def _lin_in(x, W, b):
    def body(x_ref, w_ref, b_ref, o_ref):
        res = jnp.tanh(
            jnp.dot(x_ref[...], w_ref[...], preferred_element_type=jnp.float32)
            + b_ref[...])
        _split(o_ref, res)

    return pl.pallas_call(
        body,
        grid=(N // _BLK,),
        in_specs=[
            pl.BlockSpec((_BLK, D), lambda i: (i, 0)),
            pl.BlockSpec((D, D), lambda i: (0, 0)),
            pl.BlockSpec((1, D), lambda i: (0, 0)),
        ],
        out_specs=pl.BlockSpec((NC, _BLK, H), lambda i: (0, i, 0)),
        out_shape=jax.ShapeDtypeStruct((NC, N, H), jnp.float32),
    )(x, W, b.reshape(1, D))


def _layer(agg2, h2, Wr, Ws, b):
    def body(a_ref, h_ref, wr_ref, ws_ref, b_ref, o_ref):
        a = jnp.concatenate([a_ref[0], a_ref[1]], axis=1)
        h = jnp.concatenate([h_ref[0], h_ref[1]], axis=1)
        res = jnp.tanh(
            jnp.dot(a, wr_ref[...], preferred_element_type=jnp.float32)
            + jnp.dot(h, ws_ref[...], preferred_element_type=jnp.float32)
            + b_ref[...])
        _split(o_ref, res)

    return pl.pallas_call(
        body,
        grid=(N // _BLK,),
        in_specs=[
            pl.BlockSpec((NC, _BLK, H), lambda i: (0, i, 0)),
            pl.BlockSpec((NC, _BLK, H), lambda i: (0, i, 0)),
            pl.BlockSpec((D, D), lambda i: (0, 0)),
            pl.BlockSpec((D, D), lambda i: (0, 0)),
            pl.BlockSpec((1, D), lambda i: (0, 0)),
        ],
        out_specs=pl.BlockSpec((NC, _BLK, H), lambda i: (0, i, 0)),
        out_shape=jax.ShapeDtypeStruct((NC, N, H), jnp.float32),
    )(agg2, h2, Wr, Ws, b.reshape(1, D))


def _lin_out(h2, W, b):
    def body(h_ref, w_ref, b_ref, o_ref):
        h = jnp.concatenate([h_ref[0], h_ref[1]], axis=1)
        o_ref[...] = jnp.maximum(
            jnp.dot(h, w_ref[...], preferred_element_type=jnp.float32)
            + b_ref[...], 0.0)

    return pl.pallas_call(
        body,
        grid=(N // _BLK,),
        in_specs=[
            pl.BlockSpec((NC, _BLK, H), lambda i: (0, i, 0)),
            pl.BlockSpec((D, D), lambda i: (0, 0)),
            pl.BlockSpec((1, D), lambda i: (0, 0)),
        ],
        out_specs=pl.BlockSpec((_BLK, D), lambda i: (i, 0)),
        out_shape=jax.ShapeDtypeStruct((N, D), jnp.float32),
    )(h2, W, b.reshape(1, D))


def kernel(x, edge_index, lin1_W, lin1_b,
           g1_Wr, g1_Ws, g1_b,
           g2_Wr, g2_Ws, g2_b,
           g3_Wr, g3_Ws, g3_b,
           g4_Wr, g4_Ws, g4_b,
           lin2_W, lin2_b):
    pad = EPT_PAD - EPT
    src_p = jnp.pad(edge_index[0].reshape(NS, EPT),
                    ((0, 0), (0, pad))).reshape(NS, IDXC, CHUNK)
    dst_p = jnp.pad(edge_index[1].reshape(NS, EPT),
                    ((0, 0), (0, pad)),
                    constant_values=N).reshape(NS, IDXC, CHUNK)

    h2 = _lin_in(x, lin1_W, lin1_b)
    for Wr, Ws, b in ((g1_Wr, g1_Ws, g1_b), (g2_Wr, g2_Ws, g2_b),
                      (g3_Wr, g3_Ws, g3_b), (g4_Wr, g4_Ws, g4_b)):
        agg2 = _agg_call(h2, src_p, dst_p)
        h2 = _layer(agg2, h2, Wr, Ws, b)
    return _lin_out(h2, lin2_W, lin2_b)


# speedup vs baseline: 1.4049x; 1.4049x over previous
import functools

import jax
import jax.numpy as jnp
from jax import lax
from jax.experimental import pallas as pl
from jax.experimental.pallas import tpu as pltpu
from jax.experimental.pallas import tpu_sc as plsc

N = 10000
E = 320000
D = 128
H = 64
NC = 2
NS = 16
CHUNK = 128
EPT = E // NS
CPT = 158
IDXC = CPT + 2
EPT_PAD = IDXC * CHUNK
NPAD = 10016
RPT = NPAD // NS
OPT = 624
ZCOPIES = (RPT + CHUNK - 1) // CHUNK

_BLK = 2000


def _agg_call(h2, src_p, dst_p):
    mesh = plsc.VectorSubcoreMesh(core_axis_name="c", subcore_axis_name="s")

    @functools.partial(
        pl.kernel,
        mesh=mesh,
        compiler_params=pltpu.CompilerParams(use_tc_tiling_on_sc=False),
        out_type=jax.ShapeDtypeStruct((NC, N, H), jnp.float32),
        scratch_types=[
            pltpu.VMEM((IDXC, CHUNK), jnp.int32),
            pltpu.VMEM((IDXC, CHUNK), jnp.int32),
            [pltpu.VMEM((CHUNK, H), jnp.float32)] * 2,
            pltpu.VMEM((CHUNK, H), jnp.float32),
            pltpu.VMEM_SHARED((NPAD, H), jnp.float32),
            [pltpu.SemaphoreType.DMA] * 4,
        ],
    )
    def agg_kernel(h_hbm, src_hbm, dst_hbm, out_hbm,
                   src_v, dst_v, rows_bufs, zbuf, agg_sh, sems):
        c = lax.axis_index("c")
        s = lax.axis_index("s")

        pltpu.sync_copy(src_hbm.at[s], src_v)
        pltpu.sync_copy(dst_hbm.at[s], dst_v)

        zero = jnp.zeros((16,), jnp.float32)

        def zrow(i, _):
            for l in range(H // 16):
                zbuf[i, pl.ds(l * 16, 16)] = zero
            return ()

        lax.fori_loop(0, CHUNK, zrow, ())

        zbase = s * RPT
        for k in range(ZCOPIES):
            nrows = min(CHUNK, RPT - k * CHUNK)
            pltpu.sync_copy(zbuf.at[pl.ds(0, nrows)],
                            agg_sh.at[pl.ds(zbase + k * CHUNK, nrows)])
        plsc.subcore_barrier()

        ra, rb = rows_bufs
        pltpu.async_copy(h_hbm.at[c].at[src_v.at[0]], ra, sems[0]).wait()

        def body(g, _):
            j = 2 * g
            sa = pltpu.async_copy(ra, agg_sh.at[dst_v.at[j]], sems[1],
                                  add=True)
            gb = pltpu.async_copy(h_hbm.at[c].at[src_v.at[j + 1]], rb,
                                  sems[2])
            sa.wait()
            gb.wait()
            sb = pltpu.async_copy(rb, agg_sh.at[dst_v.at[j + 1]], sems[3],
                                  add=True)
            ga = pltpu.async_copy(h_hbm.at[c].at[src_v.at[j + 2]], ra,
                                  sems[0])
            sb.wait()
            ga.wait()
            return ()

        lax.fori_loop(0, CPT // 2, body, ())
        plsc.subcore_barrier()

        obase = s * OPT
        pltpu.sync_copy(agg_sh.at[pl.ds(obase, OPT)],
                        out_hbm.at[c, pl.ds(obase, OPT)])

        @pl.when(s == NS - 1)
        def _tail():
            pltpu.sync_copy(agg_sh.at[pl.ds(NS * OPT, N - NS * OPT)],
                            out_hbm.at[c, pl.ds(NS * OPT, N - NS * OPT)])

    return agg_kernel(h2, src_p, dst_p)


def _split(o_ref, res):
    o_ref[0] = res[:, :H]
    o_ref[1] = res[:, H:]


def _lin_in(x, W, b):
    def body(x_ref, w_ref, b_ref, o_ref):
        res = jnp.tanh(
            jnp.dot(x_ref[...], w_ref[...], preferred_element_type=jnp.float32)
            + b_ref[...])
        _split(o_ref, res)

    return pl.pallas_call(
        body,
        grid=(N // _BLK,),
        in_specs=[
            pl.BlockSpec((_BLK, D), lambda i: (i, 0)),
            pl.BlockSpec((D, D), lambda i: (0, 0)),
            pl.BlockSpec((1, D), lambda i: (0, 0)),
        ],
        out_specs=pl.BlockSpec((NC, _BLK, H), lambda i: (0, i, 0)),
        out_shape=jax.ShapeDtypeStruct((NC, N, H), jnp.float32),
    )(x, W, b.reshape(1, D))


def _layer(agg2, h2, Wr, Ws, b):
    def body(a_ref, h_ref, wr_ref, ws_ref, b_ref, o_ref):
        a = jnp.concatenate([a_ref[0], a_ref[1]], axis=1)
        h = jnp.concatenate([h_ref[0], h_ref[1]], axis=1)
        res = jnp.tanh(
            jnp.dot(a, wr_ref[...], preferred_element_type=jnp.float32)
            + jnp.dot(h, ws_ref[...], preferred_element_type=jnp.float32)
            + b_ref[...])
        _split(o_ref, res)

    return pl.pallas_call(
        body,
        grid=(N // _BLK,),
        in_specs=[
            pl.BlockSpec((NC, _BLK, H), lambda i: (0, i, 0)),
            pl.BlockSpec((NC, _BLK, H), lambda i: (0, i, 0)),
            pl.BlockSpec((D, D), lambda i: (0, 0)),
            pl.BlockSpec((D, D), lambda i: (0, 0)),
            pl.BlockSpec((1, D), lambda i: (0, 0)),
        ],
        out_specs=pl.BlockSpec((NC, _BLK, H), lambda i: (0, i, 0)),
        out_shape=jax.ShapeDtypeStruct((NC, N, H), jnp.float32),
    )(agg2, h2, Wr, Ws, b.reshape(1, D))


def _lin_out(h2, W, b):
    def body(h_ref, w_ref, b_ref, o_ref):
        h = jnp.concatenate([h_ref[0], h_ref[1]], axis=1)
        o_ref[...] = jnp.maximum(
            jnp.dot(h, w_ref[...], preferred_element_type=jnp.float32)
            + b_ref[...], 0.0)

    return pl.pallas_call(
        body,
        grid=(N // _BLK,),
        in_specs=[
            pl.BlockSpec((NC, _BLK, H), lambda i: (0, i, 0)),
            pl.BlockSpec((D, D), lambda i: (0, 0)),
            pl.BlockSpec((1, D), lambda i: (0, 0)),
        ],
        out_specs=pl.BlockSpec((_BLK, D), lambda i: (i, 0)),
        out_shape=jax.ShapeDtypeStruct((N, D), jnp.float32),
    )(h2, W, b.reshape(1, D))


def kernel(x, edge_index, lin1_W, lin1_b,
           g1_Wr, g1_Ws, g1_b,
           g2_Wr, g2_Ws, g2_b,
           g3_Wr, g3_Ws, g3_b,
           g4_Wr, g4_Ws, g4_b,
           lin2_W, lin2_b):
    pad = EPT_PAD - EPT
    src_p = jnp.pad(edge_index[0].reshape(NS, EPT),
                    ((0, 0), (0, pad))).reshape(NS, IDXC, CHUNK)
    dst_p = jnp.pad(edge_index[1].reshape(NS, EPT),
                    ((0, 0), (0, pad)),
                    constant_values=N).reshape(NS, IDXC, CHUNK)

    h2 = _lin_in(x, lin1_W, lin1_b)
    for Wr, Ws, b in ((g1_Wr, g1_Ws, g1_b), (g2_Wr, g2_Ws, g2_b),
                      (g3_Wr, g3_Ws, g3_b), (g4_Wr, g4_Ws, g4_b)):
        agg2 = _agg_call(h2, src_p, dst_p)
        h2 = _layer(agg2, h2, Wr, Ws, b)
    return _lin_out(h2, lin2_W, lin2_b)


# speedup vs baseline: 1.7125x; 1.2190x over previous
import functools

import jax
import jax.numpy as jnp
from jax import lax
from jax.experimental import pallas as pl
from jax.experimental.pallas import tpu as pltpu
from jax.experimental.pallas import tpu_sc as plsc

N = 10000
E = 320000
D = 128
H = 64
NC = 2
NS = 16
CHUNK = 128
EPT = E // NS
CPT = 158
IDXC = CPT + 2
EPT_PAD = IDXC * CHUNK
NPAD = 10016
RPT = NPAD // NS
OPT = 624
ZCOPIES = (RPT + CHUNK - 1) // CHUNK

_BLK = 2000


def _agg_call(h2, sd_p):
    mesh = plsc.VectorSubcoreMesh(core_axis_name="c", subcore_axis_name="s")

    @functools.partial(
        pl.kernel,
        mesh=mesh,
        compiler_params=pltpu.CompilerParams(use_tc_tiling_on_sc=False),
        out_type=jax.ShapeDtypeStruct((NC, N, H), jnp.float32),
        scratch_types=[
            pltpu.VMEM((IDXC, CHUNK), jnp.int32),
            pltpu.VMEM((1, CHUNK), jnp.int32),
            pltpu.VMEM((1, CHUNK), jnp.int32),
            [pltpu.VMEM((CHUNK, H), jnp.float32)] * 2,
            pltpu.VMEM((CHUNK, H), jnp.float32),
            pltpu.VMEM_SHARED((NPAD, H), jnp.float32),
            pltpu.VMEM_SHARED((N, H), jnp.float32),
            [pltpu.SemaphoreType.DMA] * 4,
        ],
    )
    def agg_kernel(h_hbm, sd_hbm, out_hbm,
                   sd_v, src_row, dst_row, rows_bufs, zbuf, agg_sh, hcp_sh,
                   sems):
        c = lax.axis_index("c")
        s = lax.axis_index("s")

        pltpu.sync_copy(sd_hbm.at[s], sd_v)

        zero = jnp.zeros((16,), jnp.float32)

        def zrow(i, _):
            for l in range(H // 16):
                zbuf[i, pl.ds(l * 16, 16)] = zero
            return ()

        lax.fori_loop(0, CHUNK, zrow, ())

        zbase = s * RPT
        for k in range(ZCOPIES):
            nrows = min(CHUNK, RPT - k * CHUNK)
            pltpu.sync_copy(zbuf.at[pl.ds(0, nrows)],
                            agg_sh.at[pl.ds(zbase + k * CHUNK, nrows)])
        fbase = s * OPT
        pltpu.sync_copy(h_hbm.at[c, pl.ds(fbase, OPT)],
                        hcp_sh.at[pl.ds(fbase, OPT)])

        @pl.when(s == NS - 1)
        def _ftail():
            pltpu.sync_copy(h_hbm.at[c, pl.ds(NS * OPT, N - NS * OPT)],
                            hcp_sh.at[pl.ds(NS * OPT, N - NS * OPT)])
        plsc.subcore_barrier()

        ra, rb = rows_bufs

        def body(j, _):
            for l in range(CHUNK // 16):
                packed = sd_v[j, pl.ds(l * 16, 16)]
                src_row[0, pl.ds(l * 16, 16)] = packed >> 14
                dst_row[0, pl.ds(l * 16, 16)] = packed & 16383
            pltpu.async_copy(hcp_sh.at[src_row.at[0]], ra, sems[0]).wait()
            pltpu.sync_copy(ra, agg_sh.at[dst_row.at[0]], add=True)
            return ()

        lax.fori_loop(0, CPT, body, ())
        plsc.subcore_barrier()

        obase = s * OPT
        pltpu.sync_copy(agg_sh.at[pl.ds(obase, OPT)],
                        out_hbm.at[c, pl.ds(obase, OPT)])

        @pl.when(s == NS - 1)
        def _tail():
            pltpu.sync_copy(agg_sh.at[pl.ds(NS * OPT, N - NS * OPT)],
                            out_hbm.at[c, pl.ds(NS * OPT, N - NS * OPT)])

    return agg_kernel(h2, sd_p)


def _split(o_ref, res):
    o_ref[0] = res[:, :H]
    o_ref[1] = res[:, H:]


def _lin_in(x, W, b):
    def body(x_ref, w_ref, b_ref, o_ref):
        res = jnp.tanh(
            jnp.dot(x_ref[...], w_ref[...], preferred_element_type=jnp.float32)
            + b_ref[...])
        _split(o_ref, res)

    return pl.pallas_call(
        body,
        grid=(N // _BLK,),
        in_specs=[
            pl.BlockSpec((_BLK, D), lambda i: (i, 0)),
            pl.BlockSpec((D, D), lambda i: (0, 0)),
            pl.BlockSpec((1, D), lambda i: (0, 0)),
        ],
        out_specs=pl.BlockSpec((NC, _BLK, H), lambda i: (0, i, 0)),
        out_shape=jax.ShapeDtypeStruct((NC, N, H), jnp.float32),
    )(x, W, b.reshape(1, D))


def _layer(agg2, h2, Wr, Ws, b):
    def body(a_ref, h_ref, wr_ref, ws_ref, b_ref, o_ref):
        a = jnp.concatenate([a_ref[0], a_ref[1]], axis=1)
        h = jnp.concatenate([h_ref[0], h_ref[1]], axis=1)
        res = jnp.tanh(
            jnp.dot(a, wr_ref[...], preferred_element_type=jnp.float32)
            + jnp.dot(h, ws_ref[...], preferred_element_type=jnp.float32)
            + b_ref[...])
        _split(o_ref, res)

    return pl.pallas_call(
        body,
        grid=(N // _BLK,),
        in_specs=[
            pl.BlockSpec((NC, _BLK, H), lambda i: (0, i, 0)),
            pl.BlockSpec((NC, _BLK, H), lambda i: (0, i, 0)),
            pl.BlockSpec((D, D), lambda i: (0, 0)),
            pl.BlockSpec((D, D), lambda i: (0, 0)),
            pl.BlockSpec((1, D), lambda i: (0, 0)),
        ],
        out_specs=pl.BlockSpec((NC, _BLK, H), lambda i: (0, i, 0)),
        out_shape=jax.ShapeDtypeStruct((NC, N, H), jnp.float32),
    )(agg2, h2, Wr, Ws, b.reshape(1, D))


def _lin_out(h2, W, b):
    def body(h_ref, w_ref, b_ref, o_ref):
        h = jnp.concatenate([h_ref[0], h_ref[1]], axis=1)
        o_ref[...] = jnp.maximum(
            jnp.dot(h, w_ref[...], preferred_element_type=jnp.float32)
            + b_ref[...], 0.0)

    return pl.pallas_call(
        body,
        grid=(N // _BLK,),
        in_specs=[
            pl.BlockSpec((NC, _BLK, H), lambda i: (0, i, 0)),
            pl.BlockSpec((D, D), lambda i: (0, 0)),
            pl.BlockSpec((1, D), lambda i: (0, 0)),
        ],
        out_specs=pl.BlockSpec((_BLK, D), lambda i: (i, 0)),
        out_shape=jax.ShapeDtypeStruct((N, D), jnp.float32),
    )(h2, W, b.reshape(1, D))


def kernel(x, edge_index, lin1_W, lin1_b,
           g1_Wr, g1_Ws, g1_b,
           g2_Wr, g2_Ws, g2_b,
           g3_Wr, g3_Ws, g3_b,
           g4_Wr, g4_Ws, g4_b,
           lin2_W, lin2_b):
    pad = EPT_PAD - EPT
    sd = edge_index[0].astype(jnp.int32) * 16384 + edge_index[1]
    sd_p = jnp.pad(sd.reshape(NS, EPT), ((0, 0), (0, pad)),
                   constant_values=N).reshape(NS, IDXC, CHUNK)

    h2 = _lin_in(x, lin1_W, lin1_b)
    for Wr, Ws, b in ((g1_Wr, g1_Ws, g1_b), (g2_Wr, g2_Ws, g2_b),
                      (g3_Wr, g3_Ws, g3_b), (g4_Wr, g4_Ws, g4_b)):
        agg2 = _agg_call(h2, sd_p)
        h2 = _layer(agg2, h2, Wr, Ws, b)
    return _lin_out(h2, lin2_W, lin2_b)


# speedup vs baseline: 2.3031x; 1.3449x over previous
import functools

import jax
import jax.numpy as jnp
from jax import lax
from jax.experimental import pallas as pl
from jax.experimental.pallas import tpu as pltpu
from jax.experimental.pallas import tpu_sc as plsc

N = 10000
E = 320000
D = 128
H = 64
NC = 2
NS = 16
CHUNK = 128
EPT = E // NS
CPT = 158
IDXC = CPT + 2
EPT_PAD = IDXC * CHUNK
NPAD = 10016
RPT = NPAD // NS
OPT = 624
ZCOPIES = (RPT + CHUNK - 1) // CHUNK

_BLK = 2000


def _agg_call(h2, sd_p):
    mesh = plsc.VectorSubcoreMesh(core_axis_name="c", subcore_axis_name="s")

    @functools.partial(
        pl.kernel,
        mesh=mesh,
        compiler_params=pltpu.CompilerParams(use_tc_tiling_on_sc=False),
        out_type=jax.ShapeDtypeStruct((NC, N, H), jnp.float32),
        scratch_types=[
            pltpu.VMEM((IDXC, CHUNK), jnp.int32),
            pltpu.VMEM((1, CHUNK), jnp.int32),
            pltpu.VMEM((1, CHUNK), jnp.int32),
            pltpu.VMEM((1, CHUNK), jnp.int32),
            pltpu.VMEM((1, CHUNK), jnp.int32),
            [pltpu.VMEM((CHUNK, H), jnp.float32)] * 2,
            pltpu.VMEM((CHUNK, H), jnp.float32),
            pltpu.VMEM_SHARED((NPAD, H), jnp.float32),
            pltpu.VMEM_SHARED((N, H), jnp.float32),
            [pltpu.SemaphoreType.DMA] * 4,
        ],
    )
    def agg_kernel(h_hbm, sd_hbm, out_hbm,
                   sd_v, src_row, dst_row, src_row2, dst_row2, rows_bufs,
                   zbuf, agg_sh, hcp_sh, sems):
        c = lax.axis_index("c")
        s = lax.axis_index("s")

        pltpu.sync_copy(sd_hbm.at[s], sd_v)

        zero = jnp.zeros((16,), jnp.float32)

        def zrow(i, _):
            for l in range(H // 16):
                zbuf[i, pl.ds(l * 16, 16)] = zero
            return ()

        lax.fori_loop(0, CHUNK, zrow, ())

        zbase = s * RPT
        for k in range(ZCOPIES):
            nrows = min(CHUNK, RPT - k * CHUNK)
            pltpu.sync_copy(zbuf.at[pl.ds(0, nrows)],
                            agg_sh.at[pl.ds(zbase + k * CHUNK, nrows)])
        fbase = s * OPT
        pltpu.sync_copy(h_hbm.at[c, pl.ds(fbase, OPT)],
                        hcp_sh.at[pl.ds(fbase, OPT)])

        @pl.when(s == NS - 1)
        def _ftail():
            pltpu.sync_copy(h_hbm.at[c, pl.ds(NS * OPT, N - NS * OPT)],
                            hcp_sh.at[pl.ds(NS * OPT, N - NS * OPT)])
        plsc.subcore_barrier()

        ra, rb = rows_bufs

        def unpack(j, sref, dref):
            for l in range(CHUNK // 16):
                packed = sd_v[j, pl.ds(l * 16, 16)]
                sref[0, pl.ds(l * 16, 16)] = packed >> 14
                dref[0, pl.ds(l * 16, 16)] = packed & 16383

        unpack(0, src_row, dst_row)
        pltpu.async_copy(hcp_sh.at[src_row.at[0]], ra, sems[0]).wait()

        def body(g, _):
            j = 2 * g
            sa = pltpu.async_copy(ra, agg_sh.at[dst_row.at[0]], sems[1],
                                  add=True)
            unpack(j + 1, src_row2, dst_row2)
            gb = pltpu.async_copy(hcp_sh.at[src_row2.at[0]], rb, sems[2])
            sa.wait()
            gb.wait()
            sb = pltpu.async_copy(rb, agg_sh.at[dst_row2.at[0]], sems[3],
                                  add=True)
            unpack(j + 2, src_row, dst_row)
            ga = pltpu.async_copy(hcp_sh.at[src_row.at[0]], ra, sems[0])
            sb.wait()
            ga.wait()
            return ()

        lax.fori_loop(0, CPT // 2, body, ())
        plsc.subcore_barrier()

        obase = s * OPT
        pltpu.sync_copy(agg_sh.at[pl.ds(obase, OPT)],
                        out_hbm.at[c, pl.ds(obase, OPT)])

        @pl.when(s == NS - 1)
        def _tail():
            pltpu.sync_copy(agg_sh.at[pl.ds(NS * OPT, N - NS * OPT)],
                            out_hbm.at[c, pl.ds(NS * OPT, N - NS * OPT)])

    return agg_kernel(h2, sd_p)


def _split(o_ref, res):
    o_ref[0] = res[:, :H]
    o_ref[1] = res[:, H:]


def _lin_in(x, W, b):
    def body(x_ref, w_ref, b_ref, o_ref):
        res = jnp.tanh(
            jnp.dot(x_ref[...], w_ref[...], preferred_element_type=jnp.float32)
            + b_ref[...])
        _split(o_ref, res)

    return pl.pallas_call(
        body,
        grid=(N // _BLK,),
        in_specs=[
            pl.BlockSpec((_BLK, D), lambda i: (i, 0)),
            pl.BlockSpec((D, D), lambda i: (0, 0)),
            pl.BlockSpec((1, D), lambda i: (0, 0)),
        ],
        out_specs=pl.BlockSpec((NC, _BLK, H), lambda i: (0, i, 0)),
        out_shape=jax.ShapeDtypeStruct((NC, N, H), jnp.float32),
    )(x, W, b.reshape(1, D))


def _layer(agg2, h2, Wr, Ws, b):
    def body(a_ref, h_ref, wr_ref, ws_ref, b_ref, o_ref):
        a = jnp.concatenate([a_ref[0], a_ref[1]], axis=1)
        h = jnp.concatenate([h_ref[0], h_ref[1]], axis=1)
        res = jnp.tanh(
            jnp.dot(a, wr_ref[...], preferred_element_type=jnp.float32)
            + jnp.dot(h, ws_ref[...], preferred_element_type=jnp.float32)
            + b_ref[...])
        _split(o_ref, res)

    return pl.pallas_call(
        body,
        grid=(N // _BLK,),
        in_specs=[
            pl.BlockSpec((NC, _BLK, H), lambda i: (0, i, 0)),
            pl.BlockSpec((NC, _BLK, H), lambda i: (0, i, 0)),
            pl.BlockSpec((D, D), lambda i: (0, 0)),
            pl.BlockSpec((D, D), lambda i: (0, 0)),
            pl.BlockSpec((1, D), lambda i: (0, 0)),
        ],
        out_specs=pl.BlockSpec((NC, _BLK, H), lambda i: (0, i, 0)),
        out_shape=jax.ShapeDtypeStruct((NC, N, H), jnp.float32),
    )(agg2, h2, Wr, Ws, b.reshape(1, D))


def _lin_out(h2, W, b):
    def body(h_ref, w_ref, b_ref, o_ref):
        h = jnp.concatenate([h_ref[0], h_ref[1]], axis=1)
        o_ref[...] = jnp.maximum(
            jnp.dot(h, w_ref[...], preferred_element_type=jnp.float32)
            + b_ref[...], 0.0)

    return pl.pallas_call(
        body,
        grid=(N // _BLK,),
        in_specs=[
            pl.BlockSpec((NC, _BLK, H), lambda i: (0, i, 0)),
            pl.BlockSpec((D, D), lambda i: (0, 0)),
            pl.BlockSpec((1, D), lambda i: (0, 0)),
        ],
        out_specs=pl.BlockSpec((_BLK, D), lambda i: (i, 0)),
        out_shape=jax.ShapeDtypeStruct((N, D), jnp.float32),
    )(h2, W, b.reshape(1, D))


def kernel(x, edge_index, lin1_W, lin1_b,
           g1_Wr, g1_Ws, g1_b,
           g2_Wr, g2_Ws, g2_b,
           g3_Wr, g3_Ws, g3_b,
           g4_Wr, g4_Ws, g4_b,
           lin2_W, lin2_b):
    pad = EPT_PAD - EPT
    sd = edge_index[0].astype(jnp.int32) * 16384 + edge_index[1]
    sd_p = jnp.pad(sd.reshape(NS, EPT), ((0, 0), (0, pad)),
                   constant_values=N).reshape(NS, IDXC, CHUNK)

    h2 = _lin_in(x, lin1_W, lin1_b)
    for Wr, Ws, b in ((g1_Wr, g1_Ws, g1_b), (g2_Wr, g2_Ws, g2_b),
                      (g3_Wr, g3_Ws, g3_b), (g4_Wr, g4_Ws, g4_b)):
        agg2 = _agg_call(h2, sd_p)
        h2 = _layer(agg2, h2, Wr, Ws, b)
    return _lin_out(h2, lin2_W, lin2_b)


# speedup vs baseline: 2.3318x; 1.0125x over previous
import functools

import jax
import jax.numpy as jnp
from jax import lax
from jax.experimental import pallas as pl
from jax.experimental.pallas import tpu as pltpu
from jax.experimental.pallas import tpu_sc as plsc

N = 10000
E = 320000
D = 128
H = 64
NC = 2
NS = 16
CHUNK = 128
EPT = E // NS
CPT = 158
IDXC = CPT + 2
EPT_PAD = IDXC * CHUNK
NPAD = 10016
RPT = NPAD // NS
OPT = 624
ZCOPIES = (RPT + CHUNK - 1) // CHUNK

_BLK = 2000


def _agg_call(h2, sd_p):
    mesh = plsc.VectorSubcoreMesh(core_axis_name="c", subcore_axis_name="s")

    @functools.partial(
        pl.kernel,
        mesh=mesh,
        compiler_params=pltpu.CompilerParams(use_tc_tiling_on_sc=False),
        out_type=jax.ShapeDtypeStruct((NC, N, H), jnp.float32),
        scratch_types=[
            pltpu.VMEM((IDXC, CHUNK), jnp.int32),
            pltpu.VMEM((1, CHUNK), jnp.int32),
            pltpu.VMEM((1, CHUNK), jnp.int32),
            pltpu.VMEM((1, CHUNK), jnp.int32),
            pltpu.VMEM((1, CHUNK), jnp.int32),
            [pltpu.VMEM((CHUNK, H), jnp.float32)] * 2,
            pltpu.VMEM((CHUNK, H), jnp.float32),
            pltpu.VMEM_SHARED((NPAD, H), jnp.float32),
            pltpu.VMEM_SHARED((N, H), jnp.float32),
            [pltpu.SemaphoreType.DMA] * 4,
        ],
    )
    def agg_kernel(h_hbm, sd_hbm, out_hbm,
                   sd_v, src_row, dst_row, src_row2, dst_row2, rows_bufs,
                   zbuf, agg_sh, hcp_sh, sems):
        c = lax.axis_index("c")
        s = lax.axis_index("s")

        pltpu.sync_copy(sd_hbm.at[s], sd_v)

        zero = jnp.zeros((16,), jnp.float32)

        def zrow(i, _):
            for l in range(H // 16):
                zbuf[i, pl.ds(l * 16, 16)] = zero
            return ()

        lax.fori_loop(0, CHUNK, zrow, ())

        zbase = s * RPT
        for k in range(ZCOPIES):
            nrows = min(CHUNK, RPT - k * CHUNK)
            pltpu.sync_copy(zbuf.at[pl.ds(0, nrows)],
                            agg_sh.at[pl.ds(zbase + k * CHUNK, nrows)])
        fbase = s * OPT
        pltpu.sync_copy(h_hbm.at[c, pl.ds(fbase, OPT)],
                        hcp_sh.at[pl.ds(fbase, OPT)])

        @pl.when(s == NS - 1)
        def _ftail():
            pltpu.sync_copy(h_hbm.at[c, pl.ds(NS * OPT, N - NS * OPT)],
                            hcp_sh.at[pl.ds(NS * OPT, N - NS * OPT)])
        plsc.subcore_barrier()

        ra, rb = rows_bufs

        def unpack(j, sref, dref):
            for l in range(CHUNK // 16):
                packed = sd_v[j, pl.ds(l * 16, 16)]
                sref[0, pl.ds(l * 16, 16)] = packed >> 14
                dref[0, pl.ds(l * 16, 16)] = packed & 16383

        unpack(0, src_row, dst_row)
        pltpu.async_copy(hcp_sh.at[src_row.at[0]], ra, sems[0]).wait()

        def body(g, _):
            j = 2 * g
            sa = pltpu.async_copy(ra, agg_sh.at[dst_row.at[0]], sems[1],
                                  add=True)
            unpack(j + 1, src_row2, dst_row2)
            gb = pltpu.async_copy(hcp_sh.at[src_row2.at[0]], rb, sems[2])
            sa.wait()
            gb.wait()
            sb = pltpu.async_copy(rb, agg_sh.at[dst_row2.at[0]], sems[3],
                                  add=True)
            unpack(j + 2, src_row, dst_row)
            ga = pltpu.async_copy(hcp_sh.at[src_row.at[0]], ra, sems[0])
            sb.wait()
            ga.wait()
            return ()

        lax.fori_loop(0, CPT // 2, body, ())
        plsc.subcore_barrier()

        obase = s * OPT
        pltpu.sync_copy(agg_sh.at[pl.ds(obase, OPT)],
                        out_hbm.at[c, pl.ds(obase, OPT)])

        @pl.when(s == NS - 1)
        def _tail():
            pltpu.sync_copy(agg_sh.at[pl.ds(NS * OPT, N - NS * OPT)],
                            out_hbm.at[c, pl.ds(NS * OPT, N - NS * OPT)])

    return agg_kernel(h2, sd_p)


def _split(o_ref, res):
    o_ref[0] = res[:, :H]
    o_ref[1] = res[:, H:]


def _lin_in(x, W, b):
    def body(x_ref, w_ref, b_ref, o_ref):
        res = jnp.tanh(
            jnp.dot(x_ref[...], w_ref[...], preferred_element_type=jnp.float32)
            + b_ref[...])
        _split(o_ref, res)

    return pl.pallas_call(
        body,
        grid=(N // _BLK,),
        in_specs=[
            pl.BlockSpec((_BLK, D), lambda i: (i, 0)),
            pl.BlockSpec((D, D), lambda i: (0, 0)),
            pl.BlockSpec((1, D), lambda i: (0, 0)),
        ],
        out_specs=pl.BlockSpec((NC, _BLK, H), lambda i: (0, i, 0)),
        out_shape=jax.ShapeDtypeStruct((NC, N, H), jnp.float32),
    )(x, W, b.reshape(1, D))


def _hws(h2, Ws, b):
    def body(h_ref, ws_ref, b_ref, o_ref):
        h = jnp.concatenate([h_ref[0], h_ref[1]], axis=1)
        o_ref[...] = (
            jnp.dot(h, ws_ref[...], preferred_element_type=jnp.float32)
            + b_ref[...])

    return pl.pallas_call(
        body,
        grid=(N // _BLK,),
        in_specs=[
            pl.BlockSpec((NC, _BLK, H), lambda i: (0, i, 0)),
            pl.BlockSpec((D, D), lambda i: (0, 0)),
            pl.BlockSpec((1, D), lambda i: (0, 0)),
        ],
        out_specs=pl.BlockSpec((_BLK, D), lambda i: (i, 0)),
        out_shape=jax.ShapeDtypeStruct((N, D), jnp.float32),
    )(h2, Ws, b.reshape(1, D))


def _combine(agg2, hws, Wr):
    def body(a_ref, p_ref, wr_ref, o_ref):
        a = jnp.concatenate([a_ref[0], a_ref[1]], axis=1)
        res = jnp.tanh(
            jnp.dot(a, wr_ref[...], preferred_element_type=jnp.float32)
            + p_ref[...])
        _split(o_ref, res)

    return pl.pallas_call(
        body,
        grid=(N // _BLK,),
        in_specs=[
            pl.BlockSpec((NC, _BLK, H), lambda i: (0, i, 0)),
            pl.BlockSpec((_BLK, D), lambda i: (i, 0)),
            pl.BlockSpec((D, D), lambda i: (0, 0)),
        ],
        out_specs=pl.BlockSpec((NC, _BLK, H), lambda i: (0, i, 0)),
        out_shape=jax.ShapeDtypeStruct((NC, N, H), jnp.float32),
    )(agg2, hws, Wr)


def _combine_out(agg2, hws, Wr, W2, b2):
    def body(a_ref, p_ref, wr_ref, w2_ref, b2_ref, o_ref):
        a = jnp.concatenate([a_ref[0], a_ref[1]], axis=1)
        h = jnp.tanh(
            jnp.dot(a, wr_ref[...], preferred_element_type=jnp.float32)
            + p_ref[...])
        o_ref[...] = jnp.maximum(
            jnp.dot(h, w2_ref[...], preferred_element_type=jnp.float32)
            + b2_ref[...], 0.0)

    return pl.pallas_call(
        body,
        grid=(N // _BLK,),
        in_specs=[
            pl.BlockSpec((NC, _BLK, H), lambda i: (0, i, 0)),
            pl.BlockSpec((_BLK, D), lambda i: (i, 0)),
            pl.BlockSpec((D, D), lambda i: (0, 0)),
            pl.BlockSpec((D, D), lambda i: (0, 0)),
            pl.BlockSpec((1, D), lambda i: (0, 0)),
        ],
        out_specs=pl.BlockSpec((_BLK, D), lambda i: (i, 0)),
        out_shape=jax.ShapeDtypeStruct((N, D), jnp.float32),
    )(agg2, hws, Wr, W2, b2.reshape(1, D))


def kernel(x, edge_index, lin1_W, lin1_b,
           g1_Wr, g1_Ws, g1_b,
           g2_Wr, g2_Ws, g2_b,
           g3_Wr, g3_Ws, g3_b,
           g4_Wr, g4_Ws, g4_b,
           lin2_W, lin2_b):
    pad = EPT_PAD - EPT
    sd = edge_index[0].astype(jnp.int32) * 16384 + edge_index[1]
    sd_p = jnp.pad(sd.reshape(NS, EPT), ((0, 0), (0, pad)),
                   constant_values=N).reshape(NS, IDXC, CHUNK)

    layers = ((g1_Wr, g1_Ws, g1_b), (g2_Wr, g2_Ws, g2_b),
              (g3_Wr, g3_Ws, g3_b), (g4_Wr, g4_Ws, g4_b))
    h2 = _lin_in(x, lin1_W, lin1_b)
    for i, (Wr, Ws, b) in enumerate(layers):
        hws = _hws(h2, Ws, b)
        agg2 = _agg_call(h2, sd_p)
        if i < 3:
            h2 = _combine(agg2, hws, Wr)
        else:
            return _combine_out(agg2, hws, Wr, lin2_W, lin2_b)


# speedup vs baseline: 2.3871x; 1.0237x over previous
import functools

import jax
import jax.numpy as jnp
from jax import lax
from jax.experimental import pallas as pl
from jax.experimental.pallas import tpu as pltpu
from jax.experimental.pallas import tpu_sc as plsc

N = 10000
E = 320000
D = 128
H = 64
NC = 2
NS = 16
CHUNK = 128
EPT = E // NS
CPT = 158
IDXC = CPT + 2
EPT_PAD = IDXC * CHUNK
NPAD = 10016
RPT = NPAD // NS
OPT = 624
ZCOPIES = (RPT + CHUNK - 1) // CHUNK

_BLK = 2000


def _agg_call(h2, sd_p):
    mesh = plsc.VectorSubcoreMesh(core_axis_name="c", subcore_axis_name="s")

    @functools.partial(
        pl.kernel,
        mesh=mesh,
        compiler_params=pltpu.CompilerParams(use_tc_tiling_on_sc=False),
        out_type=jax.ShapeDtypeStruct((NC, N, H), jnp.float32),
        scratch_types=[
            pltpu.VMEM((IDXC, CHUNK), jnp.int32),
            pltpu.VMEM((1, CHUNK), jnp.int32),
            pltpu.VMEM((1, CHUNK), jnp.int32),
            pltpu.VMEM((1, CHUNK), jnp.int32),
            pltpu.VMEM((1, CHUNK), jnp.int32),
            [pltpu.VMEM((CHUNK, H), jnp.float32)] * 2,
            pltpu.VMEM((CHUNK, H), jnp.float32),
            pltpu.VMEM_SHARED((NPAD, H), jnp.float32),
            pltpu.VMEM_SHARED((N, H), jnp.float32),
            [pltpu.SemaphoreType.DMA] * 4,
        ],
    )
    def agg_kernel(h_hbm, sd_hbm, out_hbm,
                   sd_v, src_row, dst_row, src_row2, dst_row2, rows_bufs,
                   zbuf, agg_sh, hcp_sh, sems):
        c = lax.axis_index("c")
        s = lax.axis_index("s")

        pltpu.sync_copy(sd_hbm.at[s], sd_v)

        zero = jnp.zeros((16,), jnp.float32)

        def zrow(i, _):
            for l in range(H // 16):
                zbuf[i, pl.ds(l * 16, 16)] = zero
            return ()

        lax.fori_loop(0, CHUNK, zrow, ())

        zbase = s * RPT
        for k in range(ZCOPIES):
            nrows = min(CHUNK, RPT - k * CHUNK)
            pltpu.sync_copy(zbuf.at[pl.ds(0, nrows)],
                            agg_sh.at[pl.ds(zbase + k * CHUNK, nrows)])
        fbase = s * OPT
        pltpu.sync_copy(h_hbm.at[c, pl.ds(fbase, OPT)],
                        hcp_sh.at[pl.ds(fbase, OPT)])

        @pl.when(s == NS - 1)
        def _ftail():
            pltpu.sync_copy(h_hbm.at[c, pl.ds(NS * OPT, N - NS * OPT)],
                            hcp_sh.at[pl.ds(NS * OPT, N - NS * OPT)])
        plsc.subcore_barrier()

        ra, rb = rows_bufs

        def unpack(j, sref, dref):
            for l in range(CHUNK // 16):
                packed = sd_v[j, pl.ds(l * 16, 16)]
                sref[0, pl.ds(l * 16, 16)] = packed >> 14
                dref[0, pl.ds(l * 16, 16)] = packed & 16383

        unpack(0, src_row, dst_row)
        pltpu.async_copy(hcp_sh.at[src_row.at[0]], ra, sems[0]).wait()

        def body(g, _):
            j = 2 * g
            sa = pltpu.async_copy(ra, agg_sh.at[dst_row.at[0]], sems[1],
                                  add=True)
            unpack(j + 1, src_row2, dst_row2)
            gb = pltpu.async_copy(hcp_sh.at[src_row2.at[0]], rb, sems[2])
            gb.wait()
            sb = pltpu.async_copy(rb, agg_sh.at[dst_row2.at[0]], sems[3],
                                  add=True)
            sa.wait()
            unpack(j + 2, src_row, dst_row)
            ga = pltpu.async_copy(hcp_sh.at[src_row.at[0]], ra, sems[0])
            ga.wait()
            sb.wait()
            return ()

        lax.fori_loop(0, CPT // 2, body, ())
        plsc.subcore_barrier()

        obase = s * OPT
        pltpu.sync_copy(agg_sh.at[pl.ds(obase, OPT)],
                        out_hbm.at[c, pl.ds(obase, OPT)])

        @pl.when(s == NS - 1)
        def _tail():
            pltpu.sync_copy(agg_sh.at[pl.ds(NS * OPT, N - NS * OPT)],
                            out_hbm.at[c, pl.ds(NS * OPT, N - NS * OPT)])

    return agg_kernel(h2, sd_p)


def _split(o_ref, res):
    o_ref[0] = res[:, :H]
    o_ref[1] = res[:, H:]


def _lin_in(x, W, b):
    def body(x_ref, w_ref, b_ref, o_ref):
        res = jnp.tanh(
            jnp.dot(x_ref[...], w_ref[...], preferred_element_type=jnp.float32)
            + b_ref[...])
        _split(o_ref, res)

    return pl.pallas_call(
        body,
        grid=(N // _BLK,),
        in_specs=[
            pl.BlockSpec((_BLK, D), lambda i: (i, 0)),
            pl.BlockSpec((D, D), lambda i: (0, 0)),
            pl.BlockSpec((1, D), lambda i: (0, 0)),
        ],
        out_specs=pl.BlockSpec((NC, _BLK, H), lambda i: (0, i, 0)),
        out_shape=jax.ShapeDtypeStruct((NC, N, H), jnp.float32),
    )(x, W, b.reshape(1, D))


def _hws(h2, Ws, b):
    def body(h_ref, ws_ref, b_ref, o_ref):
        h = jnp.concatenate([h_ref[0], h_ref[1]], axis=1)
        o_ref[...] = (
            jnp.dot(h, ws_ref[...], preferred_element_type=jnp.float32)
            + b_ref[...])

    return pl.pallas_call(
        body,
        grid=(N // _BLK,),
        in_specs=[
            pl.BlockSpec((NC, _BLK, H), lambda i: (0, i, 0)),
            pl.BlockSpec((D, D), lambda i: (0, 0)),
            pl.BlockSpec((1, D), lambda i: (0, 0)),
        ],
        out_specs=pl.BlockSpec((_BLK, D), lambda i: (i, 0)),
        out_shape=jax.ShapeDtypeStruct((N, D), jnp.float32),
    )(h2, Ws, b.reshape(1, D))


def _combine(agg2, hws, Wr):
    def body(a_ref, p_ref, wr_ref, o_ref):
        a = jnp.concatenate([a_ref[0], a_ref[1]], axis=1)
        res = jnp.tanh(
            jnp.dot(a, wr_ref[...], preferred_element_type=jnp.float32)
            + p_ref[...])
        _split(o_ref, res)

    return pl.pallas_call(
        body,
        grid=(N // _BLK,),
        in_specs=[
            pl.BlockSpec((NC, _BLK, H), lambda i: (0, i, 0)),
            pl.BlockSpec((_BLK, D), lambda i: (i, 0)),
            pl.BlockSpec((D, D), lambda i: (0, 0)),
        ],
        out_specs=pl.BlockSpec((NC, _BLK, H), lambda i: (0, i, 0)),
        out_shape=jax.ShapeDtypeStruct((NC, N, H), jnp.float32),
    )(agg2, hws, Wr)


def _combine_out(agg2, hws, Wr, W2, b2):
    def body(a_ref, p_ref, wr_ref, w2_ref, b2_ref, o_ref):
        a = jnp.concatenate([a_ref[0], a_ref[1]], axis=1)
        h = jnp.tanh(
            jnp.dot(a, wr_ref[...], preferred_element_type=jnp.float32)
            + p_ref[...])
        o_ref[...] = jnp.maximum(
            jnp.dot(h, w2_ref[...], preferred_element_type=jnp.float32)
            + b2_ref[...], 0.0)

    return pl.pallas_call(
        body,
        grid=(N // _BLK,),
        in_specs=[
            pl.BlockSpec((NC, _BLK, H), lambda i: (0, i, 0)),
            pl.BlockSpec((_BLK, D), lambda i: (i, 0)),
            pl.BlockSpec((D, D), lambda i: (0, 0)),
            pl.BlockSpec((D, D), lambda i: (0, 0)),
            pl.BlockSpec((1, D), lambda i: (0, 0)),
        ],
        out_specs=pl.BlockSpec((_BLK, D), lambda i: (i, 0)),
        out_shape=jax.ShapeDtypeStruct((N, D), jnp.float32),
    )(agg2, hws, Wr, W2, b2.reshape(1, D))


def kernel(x, edge_index, lin1_W, lin1_b,
           g1_Wr, g1_Ws, g1_b,
           g2_Wr, g2_Ws, g2_b,
           g3_Wr, g3_Ws, g3_b,
           g4_Wr, g4_Ws, g4_b,
           lin2_W, lin2_b):
    pad = EPT_PAD - EPT
    sd = edge_index[0].astype(jnp.int32) * 16384 + edge_index[1]
    sd_p = jnp.pad(sd.reshape(NS, EPT), ((0, 0), (0, pad)),
                   constant_values=N).reshape(NS, IDXC, CHUNK)

    layers = ((g1_Wr, g1_Ws, g1_b), (g2_Wr, g2_Ws, g2_b),
              (g3_Wr, g3_Ws, g3_b), (g4_Wr, g4_Ws, g4_b))
    h2 = _lin_in(x, lin1_W, lin1_b)
    for i, (Wr, Ws, b) in enumerate(layers):
        hws = _hws(h2, Ws, b)
        agg2 = _agg_call(h2, sd_p)
        if i < 3:
            h2 = _combine(agg2, hws, Wr)
        else:
            return _combine_out(agg2, hws, Wr, lin2_W, lin2_b)


# speedup vs baseline: 2.3929x; 1.0024x over previous
import functools

import jax
import jax.numpy as jnp
from jax import lax
from jax.experimental import pallas as pl
from jax.experimental.pallas import tpu as pltpu
from jax.experimental.pallas import tpu_sc as plsc

N = 10000
E = 320000
D = 128
H = 64
NC = 2
NS = 16
CHUNK = 128
EPT = E // NS
CPT = 158
IDXC = CPT + 2
EPT_PAD = IDXC * CHUNK
NPAD = 10016
RPT = NPAD // NS
OPT = 624
ZCOPIES = (RPT + CHUNK - 1) // CHUNK

_BLK = 2000


def _agg_call(h2, sd_p):
    mesh = plsc.VectorSubcoreMesh(core_axis_name="c", subcore_axis_name="s")

    @functools.partial(
        pl.kernel,
        mesh=mesh,
        compiler_params=pltpu.CompilerParams(use_tc_tiling_on_sc=False),
        out_type=jax.ShapeDtypeStruct((NC, N, H), jnp.float32),
        scratch_types=[
            pltpu.VMEM((IDXC, CHUNK), jnp.int32),
            pltpu.VMEM((1, CHUNK), jnp.int32),
            pltpu.VMEM((1, CHUNK), jnp.int32),
            pltpu.VMEM((1, CHUNK), jnp.int32),
            pltpu.VMEM((1, CHUNK), jnp.int32),
            [pltpu.VMEM((CHUNK, H), jnp.float32)] * 2,
            pltpu.VMEM((CHUNK, H), jnp.float32),
            pltpu.VMEM_SHARED((NPAD, H), jnp.float32),
            pltpu.VMEM_SHARED((N, H), jnp.float32),
            [pltpu.SemaphoreType.DMA] * 4,
        ],
    )
    def agg_kernel(h_hbm, sd_hbm, out_hbm,
                   sd_v, src_row, dst_row, src_row2, dst_row2, rows_bufs,
                   zbuf, agg_sh, hcp_sh, sems):
        c = lax.axis_index("c")
        s = lax.axis_index("s")

        pltpu.sync_copy(sd_hbm.at[s], sd_v)

        zero = jnp.zeros((16,), jnp.float32)

        def zrow(i, _):
            for l in range(H // 16):
                zbuf[i, pl.ds(l * 16, 16)] = zero
            return ()

        lax.fori_loop(0, CHUNK, zrow, ())

        zbase = s * RPT
        for k in range(ZCOPIES):
            nrows = min(CHUNK, RPT - k * CHUNK)
            pltpu.sync_copy(zbuf.at[pl.ds(0, nrows)],
                            agg_sh.at[pl.ds(zbase + k * CHUNK, nrows)])
        fbase = s * OPT
        pltpu.sync_copy(h_hbm.at[c, pl.ds(fbase, OPT)],
                        hcp_sh.at[pl.ds(fbase, OPT)])

        @pl.when(s == NS - 1)
        def _ftail():
            pltpu.sync_copy(h_hbm.at[c, pl.ds(NS * OPT, N - NS * OPT)],
                            hcp_sh.at[pl.ds(NS * OPT, N - NS * OPT)])
        plsc.subcore_barrier()

        ra, rb = rows_bufs

        def unpack(j, sref, dref):
            for l in range(CHUNK // 16):
                packed = sd_v[j, pl.ds(l * 16, 16)]
                sref[0, pl.ds(l * 16, 16)] = packed >> 14
                dref[0, pl.ds(l * 16, 16)] = packed & 16383

        unpack(0, src_row, dst_row)
        pltpu.async_copy(hcp_sh.at[src_row.at[0]], ra, sems[0]).wait()

        def body(g, _):
            j = 2 * g
            sa = pltpu.async_copy(ra, agg_sh.at[dst_row.at[0]], sems[1],
                                  add=True)
            unpack(j + 1, src_row2, dst_row2)
            gb = pltpu.async_copy(hcp_sh.at[src_row2.at[0]], rb, sems[2])
            gb.wait()
            sb = pltpu.async_copy(rb, agg_sh.at[dst_row2.at[0]], sems[3],
                                  add=True)
            sa.wait()
            unpack(j + 2, src_row, dst_row)
            ga = pltpu.async_copy(hcp_sh.at[src_row.at[0]], ra, sems[0])
            ga.wait()
            sb.wait()
            return ()

        lax.fori_loop(0, CPT // 2, body, ())
        plsc.subcore_barrier()

        obase = s * OPT
        pltpu.sync_copy(agg_sh.at[pl.ds(obase, OPT)],
                        out_hbm.at[c, pl.ds(obase, OPT)])

        @pl.when(s == NS - 1)
        def _tail():
            pltpu.sync_copy(agg_sh.at[pl.ds(NS * OPT, N - NS * OPT)],
                            out_hbm.at[c, pl.ds(NS * OPT, N - NS * OPT)])

    return agg_kernel(h2, sd_p)


def _split(o_ref, res):
    o_ref[0] = res[:, :H]
    o_ref[1] = res[:, H:]


def _lin_in(x, W, b, Ws1, bg1):
    def body(x_ref, w_ref, b_ref, ws_ref, bg_ref, o_ref, p_ref):
        h = jnp.tanh(
            jnp.dot(x_ref[...], w_ref[...], preferred_element_type=jnp.float32)
            + b_ref[...])
        _split(o_ref, h)
        p_ref[...] = (
            jnp.dot(h, ws_ref[...], preferred_element_type=jnp.float32)
            + bg_ref[...])

    return pl.pallas_call(
        body,
        grid=(N // _BLK,),
        in_specs=[
            pl.BlockSpec((_BLK, D), lambda i: (i, 0)),
            pl.BlockSpec((D, D), lambda i: (0, 0)),
            pl.BlockSpec((1, D), lambda i: (0, 0)),
            pl.BlockSpec((D, D), lambda i: (0, 0)),
            pl.BlockSpec((1, D), lambda i: (0, 0)),
        ],
        out_specs=[pl.BlockSpec((NC, _BLK, H), lambda i: (0, i, 0)),
                   pl.BlockSpec((_BLK, D), lambda i: (i, 0))],
        out_shape=[jax.ShapeDtypeStruct((NC, N, H), jnp.float32),
                   jax.ShapeDtypeStruct((N, D), jnp.float32)],
    )(x, W, b.reshape(1, D), Ws1, bg1.reshape(1, D))


def _combine(agg2, hws, Wr, Wsn, bn):
    def body(a_ref, p_ref, wr_ref, ws_ref, bn_ref, o_ref, pn_ref):
        a = jnp.concatenate([a_ref[0], a_ref[1]], axis=1)
        h = jnp.tanh(
            jnp.dot(a, wr_ref[...], preferred_element_type=jnp.float32)
            + p_ref[...])
        _split(o_ref, h)
        pn_ref[...] = (
            jnp.dot(h, ws_ref[...], preferred_element_type=jnp.float32)
            + bn_ref[...])

    return pl.pallas_call(
        body,
        grid=(N // _BLK,),
        in_specs=[
            pl.BlockSpec((NC, _BLK, H), lambda i: (0, i, 0)),
            pl.BlockSpec((_BLK, D), lambda i: (i, 0)),
            pl.BlockSpec((D, D), lambda i: (0, 0)),
            pl.BlockSpec((D, D), lambda i: (0, 0)),
            pl.BlockSpec((1, D), lambda i: (0, 0)),
        ],
        out_specs=[pl.BlockSpec((NC, _BLK, H), lambda i: (0, i, 0)),
                   pl.BlockSpec((_BLK, D), lambda i: (i, 0))],
        out_shape=[jax.ShapeDtypeStruct((NC, N, H), jnp.float32),
                   jax.ShapeDtypeStruct((N, D), jnp.float32)],
    )(agg2, hws, Wr, Wsn, bn.reshape(1, D))


def _combine_out(agg2, hws, Wr, W2, b2):
    def body(a_ref, p_ref, wr_ref, w2_ref, b2_ref, o_ref):
        a = jnp.concatenate([a_ref[0], a_ref[1]], axis=1)
        h = jnp.tanh(
            jnp.dot(a, wr_ref[...], preferred_element_type=jnp.float32)
            + p_ref[...])
        o_ref[...] = jnp.maximum(
            jnp.dot(h, w2_ref[...], preferred_element_type=jnp.float32)
            + b2_ref[...], 0.0)

    return pl.pallas_call(
        body,
        grid=(N // _BLK,),
        in_specs=[
            pl.BlockSpec((NC, _BLK, H), lambda i: (0, i, 0)),
            pl.BlockSpec((_BLK, D), lambda i: (i, 0)),
            pl.BlockSpec((D, D), lambda i: (0, 0)),
            pl.BlockSpec((D, D), lambda i: (0, 0)),
            pl.BlockSpec((1, D), lambda i: (0, 0)),
        ],
        out_specs=pl.BlockSpec((_BLK, D), lambda i: (i, 0)),
        out_shape=jax.ShapeDtypeStruct((N, D), jnp.float32),
    )(agg2, hws, Wr, W2, b2.reshape(1, D))


def kernel(x, edge_index, lin1_W, lin1_b,
           g1_Wr, g1_Ws, g1_b,
           g2_Wr, g2_Ws, g2_b,
           g3_Wr, g3_Ws, g3_b,
           g4_Wr, g4_Ws, g4_b,
           lin2_W, lin2_b):
    pad = EPT_PAD - EPT
    sd = edge_index[0].astype(jnp.int32) * 16384 + edge_index[1]
    sd_p = jnp.pad(sd.reshape(NS, EPT), ((0, 0), (0, pad)),
                   constant_values=N).reshape(NS, IDXC, CHUNK)

    h2, hws = _lin_in(x, lin1_W, lin1_b, g1_Ws, g1_b)
    nxt = ((g2_Ws, g2_b), (g3_Ws, g3_b), (g4_Ws, g4_b))
    for i, Wr in enumerate((g1_Wr, g2_Wr, g3_Wr)):
        agg2 = _agg_call(h2, sd_p)
        h2, hws = _combine(agg2, hws, Wr, *nxt[i])
    agg2 = _agg_call(h2, sd_p)
    return _combine_out(agg2, hws, g4_Wr, lin2_W, lin2_b)


# speedup vs baseline: 2.6810x; 1.1204x over previous
import functools

import jax
import jax.numpy as jnp
from jax import lax
from jax.experimental import pallas as pl
from jax.experimental.pallas import tpu as pltpu
from jax.experimental.pallas import tpu_sc as plsc

N = 10000
E = 320000
D = 128
H = 64
NC = 2
NS = 16
CHUNK = 128
EPT = E // NS
CPT = 158
IDXC = CPT + 2
EPT_PAD = IDXC * CHUNK
NPAD = 10016
RPT = NPAD // NS
OPT = 624
ZCOPIES = (RPT + CHUNK - 1) // CHUNK

_BLK = 2000


def _agg_call(h, sd_p):
    mesh = plsc.VectorSubcoreMesh(core_axis_name="c", subcore_axis_name="s")

    @functools.partial(
        pl.kernel,
        mesh=mesh,
        compiler_params=pltpu.CompilerParams(use_tc_tiling_on_sc=False),
        out_type=jax.ShapeDtypeStruct((N, D), jnp.float32),
        scratch_types=[
            pltpu.VMEM((IDXC, CHUNK), jnp.int32),
            pltpu.VMEM((1, CHUNK), jnp.int32),
            pltpu.VMEM((1, CHUNK), jnp.int32),
            pltpu.VMEM((1, CHUNK), jnp.int32),
            pltpu.VMEM((1, CHUNK), jnp.int32),
            [pltpu.VMEM((CHUNK, H), jnp.float32)] * 2,
            pltpu.VMEM((CHUNK, H), jnp.float32),
            pltpu.VMEM_SHARED((NPAD, H), jnp.float32),
            pltpu.VMEM_SHARED((N, H), jnp.float32),
            [pltpu.SemaphoreType.DMA] * 4,
        ],
    )
    def agg_kernel(h_hbm, sd_hbm, out_hbm,
                   sd_v, src_row, dst_row, src_row2, dst_row2, rows_bufs,
                   zbuf, agg_sh, hcp_sh, sems):
        c = lax.axis_index("c")
        s = lax.axis_index("s")

        pltpu.sync_copy(sd_hbm.at[s], sd_v)

        zero = jnp.zeros((16,), jnp.float32)

        def zrow(i, _):
            for l in range(H // 16):
                zbuf[i, pl.ds(l * 16, 16)] = zero
            return ()

        lax.fori_loop(0, CHUNK, zrow, ())

        zbase = s * RPT
        for k in range(ZCOPIES):
            nrows = min(CHUNK, RPT - k * CHUNK)
            pltpu.sync_copy(zbuf.at[pl.ds(0, nrows)],
                            agg_sh.at[pl.ds(zbase + k * CHUNK, nrows)])
        fbase = s * OPT
        ftail = N - NS * OPT
        for half in range(NC):
            @pl.when(c == half)
            def _fill():
                pltpu.sync_copy(
                    h_hbm.at[pl.ds(fbase, OPT), pl.ds(half * H, H)],
                    hcp_sh.at[pl.ds(fbase, OPT)])

                @pl.when(s == NS - 1)
                def _ftail():
                    pltpu.sync_copy(
                        h_hbm.at[pl.ds(NS * OPT, ftail), pl.ds(half * H, H)],
                        hcp_sh.at[pl.ds(NS * OPT, ftail)])
        plsc.subcore_barrier()

        ra, rb = rows_bufs

        def unpack(j, sref, dref):
            for l in range(CHUNK // 16):
                packed = sd_v[j, pl.ds(l * 16, 16)]
                sref[0, pl.ds(l * 16, 16)] = packed >> 14
                dref[0, pl.ds(l * 16, 16)] = packed & 16383

        unpack(0, src_row, dst_row)
        pltpu.async_copy(hcp_sh.at[src_row.at[0]], ra, sems[0]).wait()

        def body(g, _):
            j = 2 * g
            sa = pltpu.async_copy(ra, agg_sh.at[dst_row.at[0]], sems[1],
                                  add=True)
            unpack(j + 1, src_row2, dst_row2)
            gb = pltpu.async_copy(hcp_sh.at[src_row2.at[0]], rb, sems[2])
            gb.wait()
            sb = pltpu.async_copy(rb, agg_sh.at[dst_row2.at[0]], sems[3],
                                  add=True)
            sa.wait()
            unpack(j + 2, src_row, dst_row)
            ga = pltpu.async_copy(hcp_sh.at[src_row.at[0]], ra, sems[0])
            ga.wait()
            sb.wait()
            return ()

        lax.fori_loop(0, CPT // 2, body, ())
        plsc.subcore_barrier()

        obase = s * OPT
        otail = N - NS * OPT
        for half in range(NC):
            @pl.when(c == half)
            def _wout():
                pltpu.sync_copy(
                    agg_sh.at[pl.ds(obase, OPT)],
                    out_hbm.at[pl.ds(obase, OPT), pl.ds(half * H, H)])

                @pl.when(s == NS - 1)
                def _tail():
                    pltpu.sync_copy(
                        agg_sh.at[pl.ds(NS * OPT, otail)],
                        out_hbm.at[pl.ds(NS * OPT, otail), pl.ds(half * H, H)])

    return agg_kernel(h, sd_p)


def _lin_in(x, W, b, Ws1, bg1):
    def body(x_ref, w_ref, b_ref, ws_ref, bg_ref, o_ref, p_ref):
        h = jnp.tanh(
            jnp.dot(x_ref[...], w_ref[...], preferred_element_type=jnp.float32)
            + b_ref[...])
        o_ref[...] = h
        p_ref[...] = (
            jnp.dot(h, ws_ref[...], preferred_element_type=jnp.float32)
            + bg_ref[...])

    return pl.pallas_call(
        body,
        grid=(N // _BLK,),
        in_specs=[
            pl.BlockSpec((_BLK, D), lambda i: (i, 0)),
            pl.BlockSpec((D, D), lambda i: (0, 0)),
            pl.BlockSpec((1, D), lambda i: (0, 0)),
            pl.BlockSpec((D, D), lambda i: (0, 0)),
            pl.BlockSpec((1, D), lambda i: (0, 0)),
        ],
        out_specs=[pl.BlockSpec((_BLK, D), lambda i: (i, 0)),
                   pl.BlockSpec((_BLK, D), lambda i: (i, 0))],
        out_shape=[jax.ShapeDtypeStruct((N, D), jnp.float32),
                   jax.ShapeDtypeStruct((N, D), jnp.float32)],
    )(x, W, b.reshape(1, D), Ws1, bg1.reshape(1, D))


def _combine(agg, hws, Wr, Wsn, bn):
    def body(a_ref, p_ref, wr_ref, ws_ref, bn_ref, o_ref, pn_ref):
        h = jnp.tanh(
            jnp.dot(a_ref[...], wr_ref[...], preferred_element_type=jnp.float32)
            + p_ref[...])
        o_ref[...] = h
        pn_ref[...] = (
            jnp.dot(h, ws_ref[...], preferred_element_type=jnp.float32)
            + bn_ref[...])

    return pl.pallas_call(
        body,
        grid=(N // _BLK,),
        in_specs=[
            pl.BlockSpec((_BLK, D), lambda i: (i, 0)),
            pl.BlockSpec((_BLK, D), lambda i: (i, 0)),
            pl.BlockSpec((D, D), lambda i: (0, 0)),
            pl.BlockSpec((D, D), lambda i: (0, 0)),
            pl.BlockSpec((1, D), lambda i: (0, 0)),
        ],
        out_specs=[pl.BlockSpec((_BLK, D), lambda i: (i, 0)),
                   pl.BlockSpec((_BLK, D), lambda i: (i, 0))],
        out_shape=[jax.ShapeDtypeStruct((N, D), jnp.float32),
                   jax.ShapeDtypeStruct((N, D), jnp.float32)],
    )(agg, hws, Wr, Wsn, bn.reshape(1, D))


def _combine_out(agg, hws, Wr, W2, b2):
    def body(a_ref, p_ref, wr_ref, w2_ref, b2_ref, o_ref):
        h = jnp.tanh(
            jnp.dot(a_ref[...], wr_ref[...], preferred_element_type=jnp.float32)
            + p_ref[...])
        o_ref[...] = jnp.maximum(
            jnp.dot(h, w2_ref[...], preferred_element_type=jnp.float32)
            + b2_ref[...], 0.0)

    return pl.pallas_call(
        body,
        grid=(N // _BLK,),
        in_specs=[
            pl.BlockSpec((_BLK, D), lambda i: (i, 0)),
            pl.BlockSpec((_BLK, D), lambda i: (i, 0)),
            pl.BlockSpec((D, D), lambda i: (0, 0)),
            pl.BlockSpec((D, D), lambda i: (0, 0)),
            pl.BlockSpec((1, D), lambda i: (0, 0)),
        ],
        out_specs=pl.BlockSpec((_BLK, D), lambda i: (i, 0)),
        out_shape=jax.ShapeDtypeStruct((N, D), jnp.float32),
    )(agg, hws, Wr, W2, b2.reshape(1, D))


def kernel(x, edge_index, lin1_W, lin1_b,
           g1_Wr, g1_Ws, g1_b,
           g2_Wr, g2_Ws, g2_b,
           g3_Wr, g3_Ws, g3_b,
           g4_Wr, g4_Ws, g4_b,
           lin2_W, lin2_b):
    pad = EPT_PAD - EPT
    sd = edge_index[0].astype(jnp.int32) * 16384 + edge_index[1]
    sd_p = jnp.pad(sd.reshape(NS, EPT), ((0, 0), (0, pad)),
                   constant_values=N).reshape(NS, IDXC, CHUNK)

    h, hws = _lin_in(x, lin1_W, lin1_b, g1_Ws, g1_b)
    nxt = ((g2_Ws, g2_b), (g3_Ws, g3_b), (g4_Ws, g4_b))
    for i, Wr in enumerate((g1_Wr, g2_Wr, g3_Wr)):
        agg = _agg_call(h, sd_p)
        h, hws = _combine(agg, hws, Wr, *nxt[i])
    agg = _agg_call(h, sd_p)
    return _combine_out(agg, hws, g4_Wr, lin2_W, lin2_b)


# speedup vs baseline: 2.7054x; 1.0091x over previous
import functools

import jax
import jax.numpy as jnp
from jax import lax
from jax.experimental import pallas as pl
from jax.experimental.pallas import tpu as pltpu
from jax.experimental.pallas import tpu_sc as plsc

N = 10000
E = 320000
D = 128
H = 64
NC = 2
NS = 16
CHUNK = 128
EPT = E // NS
CPT = 158
IDXC = CPT + 2
EPT_PAD = IDXC * CHUNK
NPAD = 10016
RPT = NPAD // NS
OPT = 624
ZCOPIES = (RPT + CHUNK - 1) // CHUNK

_BLK = 2000


def _agg_call(h, sd_p):
    mesh = plsc.VectorSubcoreMesh(core_axis_name="c", subcore_axis_name="s")

    @functools.partial(
        pl.kernel,
        mesh=mesh,
        compiler_params=pltpu.CompilerParams(use_tc_tiling_on_sc=False),
        out_type=jax.ShapeDtypeStruct((N, D), jnp.float32),
        scratch_types=[
            pltpu.VMEM((IDXC, CHUNK), jnp.int32),
            pltpu.VMEM((1, CHUNK), jnp.int32),
            pltpu.VMEM((1, CHUNK), jnp.int32),
            pltpu.VMEM((1, CHUNK), jnp.int32),
            pltpu.VMEM((1, CHUNK), jnp.int32),
            [pltpu.VMEM((CHUNK, H), jnp.float32)] * 2,
            pltpu.VMEM((CHUNK, H), jnp.float32),
            pltpu.VMEM_SHARED((NPAD, H), jnp.float32),
            pltpu.VMEM_SHARED((N, H), jnp.float32),
            [pltpu.SemaphoreType.DMA] * 4,
        ],
    )
    def agg_kernel(h_hbm, sd_hbm, out_hbm,
                   sd_v, src_row, dst_row, src_row2, dst_row2, rows_bufs,
                   zbuf, agg_sh, hcp_sh, sems):
        c = lax.axis_index("c")
        s = lax.axis_index("s")

        d_idx = pltpu.async_copy(sd_hbm.at[s], sd_v, sems[0])

        zero = jnp.zeros((16,), jnp.float32)

        def zrow(i, _):
            for l in range(H // 16):
                zbuf[i, pl.ds(l * 16, 16)] = zero
            return ()

        lax.fori_loop(0, CHUNK, zrow, ())

        fbase = s * OPT
        ftail = N - NS * OPT
        for half in range(NC):
            @pl.when(c == half)
            def _fill():
                d_f = pltpu.async_copy(
                    h_hbm.at[pl.ds(fbase, OPT), pl.ds(half * H, H)],
                    hcp_sh.at[pl.ds(fbase, OPT)], sems[1])

                @pl.when(s == NS - 1)
                def _ftail():
                    pltpu.sync_copy(
                        h_hbm.at[pl.ds(NS * OPT, ftail), pl.ds(half * H, H)],
                        hcp_sh.at[pl.ds(NS * OPT, ftail)])
                d_f.wait()

        zbase = s * RPT
        for k in range(ZCOPIES):
            nrows = min(CHUNK, RPT - k * CHUNK)
            pltpu.sync_copy(zbuf.at[pl.ds(0, nrows)],
                            agg_sh.at[pl.ds(zbase + k * CHUNK, nrows)])
        d_idx.wait()
        plsc.subcore_barrier()

        ra, rb = rows_bufs

        def unpack(j, sref, dref):
            for l in range(CHUNK // 16):
                packed = sd_v[j, pl.ds(l * 16, 16)]
                sref[0, pl.ds(l * 16, 16)] = packed >> 14
                dref[0, pl.ds(l * 16, 16)] = packed & 16383

        unpack(0, src_row, dst_row)
        pltpu.async_copy(hcp_sh.at[src_row.at[0]], ra, sems[0]).wait()

        def body(g, _):
            j = 2 * g
            sa = pltpu.async_copy(ra, agg_sh.at[dst_row.at[0]], sems[1],
                                  add=True)
            unpack(j + 1, src_row2, dst_row2)
            gb = pltpu.async_copy(hcp_sh.at[src_row2.at[0]], rb, sems[2])
            gb.wait()
            sb = pltpu.async_copy(rb, agg_sh.at[dst_row2.at[0]], sems[3],
                                  add=True)
            sa.wait()
            unpack(j + 2, src_row, dst_row)
            ga = pltpu.async_copy(hcp_sh.at[src_row.at[0]], ra, sems[0])
            ga.wait()
            sb.wait()
            return ()

        lax.fori_loop(0, CPT // 2, body, ())
        plsc.subcore_barrier()

        obase = s * OPT
        otail = N - NS * OPT
        for half in range(NC):
            @pl.when(c == half)
            def _wout():
                pltpu.sync_copy(
                    agg_sh.at[pl.ds(obase, OPT)],
                    out_hbm.at[pl.ds(obase, OPT), pl.ds(half * H, H)])

                @pl.when(s == NS - 1)
                def _tail():
                    pltpu.sync_copy(
                        agg_sh.at[pl.ds(NS * OPT, otail)],
                        out_hbm.at[pl.ds(NS * OPT, otail), pl.ds(half * H, H)])

    return agg_kernel(h, sd_p)


def _lin_in(x, W, b, Ws1, bg1):
    def body(x_ref, w_ref, b_ref, ws_ref, bg_ref, o_ref, p_ref):
        h = jnp.tanh(
            jnp.dot(x_ref[...], w_ref[...], preferred_element_type=jnp.float32)
            + b_ref[...])
        o_ref[...] = h
        p_ref[...] = (
            jnp.dot(h, ws_ref[...], preferred_element_type=jnp.float32)
            + bg_ref[...])

    return pl.pallas_call(
        body,
        grid=(N // _BLK,),
        in_specs=[
            pl.BlockSpec((_BLK, D), lambda i: (i, 0)),
            pl.BlockSpec((D, D), lambda i: (0, 0)),
            pl.BlockSpec((1, D), lambda i: (0, 0)),
            pl.BlockSpec((D, D), lambda i: (0, 0)),
            pl.BlockSpec((1, D), lambda i: (0, 0)),
        ],
        out_specs=[pl.BlockSpec((_BLK, D), lambda i: (i, 0)),
                   pl.BlockSpec((_BLK, D), lambda i: (i, 0))],
        out_shape=[jax.ShapeDtypeStruct((N, D), jnp.float32),
                   jax.ShapeDtypeStruct((N, D), jnp.float32)],
    )(x, W, b.reshape(1, D), Ws1, bg1.reshape(1, D))


def _combine(agg, hws, Wr, Wsn, bn):
    def body(a_ref, p_ref, wr_ref, ws_ref, bn_ref, o_ref, pn_ref):
        h = jnp.tanh(
            jnp.dot(a_ref[...], wr_ref[...], preferred_element_type=jnp.float32)
            + p_ref[...])
        o_ref[...] = h
        pn_ref[...] = (
            jnp.dot(h, ws_ref[...], preferred_element_type=jnp.float32)
            + bn_ref[...])

    return pl.pallas_call(
        body,
        grid=(N // _BLK,),
        in_specs=[
            pl.BlockSpec((_BLK, D), lambda i: (i, 0)),
            pl.BlockSpec((_BLK, D), lambda i: (i, 0)),
            pl.BlockSpec((D, D), lambda i: (0, 0)),
            pl.BlockSpec((D, D), lambda i: (0, 0)),
            pl.BlockSpec((1, D), lambda i: (0, 0)),
        ],
        out_specs=[pl.BlockSpec((_BLK, D), lambda i: (i, 0)),
                   pl.BlockSpec((_BLK, D), lambda i: (i, 0))],
        out_shape=[jax.ShapeDtypeStruct((N, D), jnp.float32),
                   jax.ShapeDtypeStruct((N, D), jnp.float32)],
    )(agg, hws, Wr, Wsn, bn.reshape(1, D))


def _combine_out(agg, hws, Wr, W2, b2):
    def body(a_ref, p_ref, wr_ref, w2_ref, b2_ref, o_ref):
        h = jnp.tanh(
            jnp.dot(a_ref[...], wr_ref[...], preferred_element_type=jnp.float32)
            + p_ref[...])
        o_ref[...] = jnp.maximum(
            jnp.dot(h, w2_ref[...], preferred_element_type=jnp.float32)
            + b2_ref[...], 0.0)

    return pl.pallas_call(
        body,
        grid=(N // _BLK,),
        in_specs=[
            pl.BlockSpec((_BLK, D), lambda i: (i, 0)),
            pl.BlockSpec((_BLK, D), lambda i: (i, 0)),
            pl.BlockSpec((D, D), lambda i: (0, 0)),
            pl.BlockSpec((D, D), lambda i: (0, 0)),
            pl.BlockSpec((1, D), lambda i: (0, 0)),
        ],
        out_specs=pl.BlockSpec((_BLK, D), lambda i: (i, 0)),
        out_shape=jax.ShapeDtypeStruct((N, D), jnp.float32),
    )(agg, hws, Wr, W2, b2.reshape(1, D))


def kernel(x, edge_index, lin1_W, lin1_b,
           g1_Wr, g1_Ws, g1_b,
           g2_Wr, g2_Ws, g2_b,
           g3_Wr, g3_Ws, g3_b,
           g4_Wr, g4_Ws, g4_b,
           lin2_W, lin2_b):
    pad = EPT_PAD - EPT
    sd = edge_index[0].astype(jnp.int32) * 16384 + edge_index[1]
    sd_p = jnp.pad(sd.reshape(NS, EPT), ((0, 0), (0, pad)),
                   constant_values=N).reshape(NS, IDXC, CHUNK)

    h, hws = _lin_in(x, lin1_W, lin1_b, g1_Ws, g1_b)
    nxt = ((g2_Ws, g2_b), (g3_Ws, g3_b), (g4_Ws, g4_b))
    for i, Wr in enumerate((g1_Wr, g2_Wr, g3_Wr)):
        agg = _agg_call(h, sd_p)
        h, hws = _combine(agg, hws, Wr, *nxt[i])
    agg = _agg_call(h, sd_p)
    return _combine_out(agg, hws, g4_Wr, lin2_W, lin2_b)
